# baseline (speedup 1.0000x reference)
import jax
import jax.numpy as jnp
from jax import lax
from jax.experimental import pallas as pl
from jax.experimental.pallas import tpu as pltpu
from jax.experimental.pallas import tpu_sc as plsc

_N = 10000
_E = 320000
_F = 128
_H = _F // 2
_R = 8

_NC = 2
_NS = 16
_EPT = _E // _NS
_GCH = 200
_NG = _EPT // _GCH
_SCH = 100
_NSUB = _GCH // _SCH
_NP = 10240
_RPT = _NP // _NS


def _mm_body(x_ref, w_ref, o_ref):
    o_ref[...] = jnp.dot(x_ref[...], w_ref[0],
                         preferred_element_type=jnp.float32)


def _rel_transform(x, weight):
    bn = 2000
    nb = _N // bn
    return pl.pallas_call(
        _mm_body,
        grid=(nb, _R),
        in_specs=[
            pl.BlockSpec((bn, _F), lambda i, r: (i, 0)),
            pl.BlockSpec((1, _F, _F), lambda i, r: (r, 0, 0)),
        ],
        out_specs=pl.BlockSpec((bn, _F), lambda i, r: (r * nb + i, 0)),
        out_shape=jax.ShapeDtypeStruct((_R * _N, _F), jnp.float32),
    )(x, weight)


def _gi_body(et_ref, ei_ref, o_ref):
    g = (et_ref[0] * _N + ei_ref[0]) * 2
    o_ref[0] = g
    o_ref[1] = g + 1


def _gather_indices(et_row, edge_index):
    be = 32000
    return pl.pallas_call(
        _gi_body,
        grid=(_E // be,),
        in_specs=[
            pl.BlockSpec((1, be), lambda i: (0, i)),
            pl.BlockSpec((2, be), lambda i: (0, i)),
        ],
        out_specs=pl.BlockSpec((2, be), lambda i: (0, i)),
        out_shape=jax.ShapeDtypeStruct((2, _E), jnp.int32),
    )(et_row, edge_index)


def _sc_body(xr_hbm, gi_hbm, dst_hbm, zeros_hbm, out_hbm,
             gi_v, dst_v, rows0_v, rows1_v, agg_s, gsem0, gsem1):
    c = lax.axis_index("c")
    s = lax.axis_index("s")

    pltpu.sync_copy(gi_hbm.at[c, pl.ds(s * _EPT, _EPT)], gi_v)
    pltpu.sync_copy(dst_hbm.at[s], dst_v)

    rows = pl.ds(s * _RPT, _RPT)
    pltpu.sync_copy(zeros_hbm.at[rows], agg_s.at[rows])

    plsc.subcore_barrier()

    bufs = (rows0_v, rows1_v)
    gsems = (gsem0, gsem1)

    def _gather_start(m, b):
        pltpu.async_copy(xr_hbm.at[gi_v.at[pl.ds(m * _GCH, _GCH)]],
                         bufs[b], gsems[b])

    def _gather_wait(b):
        pltpu.make_async_copy(xr_hbm.at[gi_v.at[pl.ds(0, _GCH)]],
                              bufs[b], gsems[b]).wait()

    _gather_start(0, 0)

    def _pair(t, carry):
        for b in range(2):
            m = t * 2 + b
            _gather_wait(b)
            nxt = m + 1

            @pl.when(nxt < _NG)
            def _():
                _gather_start(nxt, (b + 1) % 2)

            for k in range(_NSUB):
                pltpu.sync_copy(
                    bufs[b].at[pl.ds(k * _SCH, _SCH)],
                    agg_s.at[dst_v.at[m * _NSUB + k]], add=True)
        return carry
    lax.fori_loop(0, _NG // 2, _pair, 0)

    plsc.subcore_barrier()

    pltpu.sync_copy(agg_s.at[rows], out_hbm.at[c, rows])


_sc_scatter = pl.kernel(
    _sc_body,
    out_type=jax.ShapeDtypeStruct((_NC, _NP, _H), jnp.float32),
    mesh=plsc.VectorSubcoreMesh(core_axis_name="c", subcore_axis_name="s",
                                num_cores=_NC, num_subcores=_NS),
    scratch_types=[
        pltpu.VMEM((_EPT,), jnp.int32),
        pltpu.VMEM((_EPT // _SCH, _SCH), jnp.int32),
        pltpu.VMEM((_GCH, _H), jnp.float32),
        pltpu.VMEM((_GCH, _H), jnp.float32),
        pltpu.VMEM_SHARED((_NP, _H), jnp.float32),
        pltpu.SemaphoreType.DMA,
        pltpu.SemaphoreType.DMA,
    ],
    compiler_params=pltpu.CompilerParams(use_tc_tiling_on_sc=False),
)


def _ep_body(p_ref, b_ref, o_ref):
    full = jnp.concatenate([p_ref[0], p_ref[1]], axis=1)
    bias = jnp.concatenate([b_ref[0], b_ref[1]], axis=1)
    o_ref[...] = full + bias


def _epilogue(parts, bias2d):
    bn = 2000
    return pl.pallas_call(
        _ep_body,
        grid=(_N // bn,),
        in_specs=[
            pl.BlockSpec((_NC, bn, _H), lambda i: (0, i, 0)),
            pl.BlockSpec((_NC, 1, _H), lambda i: (0, 0, 0)),
        ],
        out_specs=pl.BlockSpec((bn, _F), lambda i: (i, 0)),
        out_shape=jax.ShapeDtypeStruct((_N, _F), jnp.float32),
    )(parts, bias2d)


def kernel(x, edge_index, edge_type, weight, h_bias):
    xr = _rel_transform(x, weight).reshape(_NC * _R * _N, _H)
    gi = _gather_indices(edge_type.reshape(1, _E), edge_index)
    zeros = jnp.zeros((_NP, _H), jnp.float32)
    dst = edge_index[1].reshape(_NS, _EPT // _SCH, _SCH)
    parts = _sc_scatter(xr, gi, dst, zeros)
    return _epilogue(parts, h_bias.reshape(_NC, 1, _H))


# speedup vs baseline: 1.2559x; 1.2559x over previous
import jax
import jax.numpy as jnp
from jax import lax
from jax.experimental import pallas as pl
from jax.experimental.pallas import tpu as pltpu
from jax.experimental.pallas import tpu_sc as plsc

_N = 10000
_E = 320000
_F = 128
_H = _F // 2
_R = 8

_NC = 2
_NS = 16
_EPT = _E // _NS
_NPH = 2
_EPH = _EPT // _NPH
_GCH = 400
_NG = _EPH // _GCH
_SCH = 100
_NSUB = _GCH // _SCH
_NP = 10240
_RPT = _NP // _NS


def _mm_body(x_ref, w_ref, o_ref):
    o_ref[...] = jnp.dot(x_ref[...], w_ref[0],
                         preferred_element_type=jnp.float32)


def _rel_transform(x, weight):
    return pl.pallas_call(
        _mm_body,
        grid=(_R,),
        in_specs=[
            pl.BlockSpec((_N, _F), lambda r: (0, 0)),
            pl.BlockSpec((1, _F, _F), lambda r: (r, 0, 0)),
        ],
        out_specs=pl.BlockSpec((_N, _F), lambda r: (r, 0)),
        out_shape=jax.ShapeDtypeStruct((_R * _N, _F), jnp.float32),
    )(x, weight)


def _gi_body(et_ref, src_ref, o_ref):
    g = (et_ref[...] * _N + src_ref[...]) * 2
    o_ref[0] = g
    o_ref[1] = g + 1


def _gather_indices(et2d, src2d):
    rows = _E // _F
    return pl.pallas_call(
        _gi_body,
        grid=(1,),
        in_specs=[
            pl.BlockSpec((rows, _F), lambda i: (0, 0)),
            pl.BlockSpec((rows, _F), lambda i: (0, 0)),
        ],
        out_specs=pl.BlockSpec((2, rows, _F), lambda i: (0, 0, 0)),
        out_shape=jax.ShapeDtypeStruct((2, rows, _F), jnp.int32),
    )(et2d, src2d)


def _sc_body(xr_hbm, gi_hbm, dst_hbm, zeros_hbm, out_hbm,
             gi_v, dst_v, rows0_v, rows1_v, agg_s, gsem0, gsem1):
    c = lax.axis_index("c")
    s = lax.axis_index("s")

    rows = pl.ds(s * _RPT, _RPT)
    pltpu.sync_copy(zeros_hbm.at[rows], agg_s.at[rows])

    plsc.subcore_barrier()

    bufs = (rows0_v, rows1_v)
    gsems = (gsem0, gsem1)

    def _gather_start(m, b):
        pltpu.async_copy(xr_hbm.at[gi_v.at[pl.ds(m * _GCH, _GCH)]],
                         bufs[b], gsems[b])

    def _gather_wait(b):
        pltpu.make_async_copy(xr_hbm.at[gi_v.at[pl.ds(0, _GCH)]],
                              bufs[b], gsems[b]).wait()

    for ph in range(_NPH):
        pltpu.sync_copy(gi_hbm.at[c, s, pl.ds(ph * _EPH, _EPH)], gi_v)
        pltpu.sync_copy(dst_hbm.at[s, ph], dst_v)

        _gather_start(0, 0)

        def _pair(t, carry):
            for b in range(2):
                m = t * 2 + b
                _gather_wait(b)
                nxt = m + 1

                @pl.when(nxt < _NG)
                def _():
                    _gather_start(nxt, (b + 1) % 2)

                for k in range(_NSUB):
                    pltpu.sync_copy(
                        bufs[b].at[pl.ds(k * _SCH, _SCH)],
                        agg_s.at[dst_v.at[m * _NSUB + k]], add=True)
            return carry
        lax.fori_loop(0, _NG // 2, _pair, 0)

        if _NG % 2:
            m = _NG - 1
            _gather_wait(0)
            for k in range(_NSUB):
                pltpu.sync_copy(
                    bufs[0].at[pl.ds(k * _SCH, _SCH)],
                    agg_s.at[dst_v.at[m * _NSUB + k]], add=True)

    plsc.subcore_barrier()

    pltpu.sync_copy(agg_s.at[rows], out_hbm.at[c, rows])


_sc_scatter = pl.kernel(
    _sc_body,
    out_type=jax.ShapeDtypeStruct((_NC, _NP, _H), jnp.float32),
    mesh=plsc.VectorSubcoreMesh(core_axis_name="c", subcore_axis_name="s",
                                num_cores=_NC, num_subcores=_NS),
    scratch_types=[
        pltpu.VMEM((_EPH,), jnp.int32),
        pltpu.VMEM((_EPH // _SCH, _SCH), jnp.int32),
        pltpu.VMEM((_GCH, _H), jnp.float32),
        pltpu.VMEM((_GCH, _H), jnp.float32),
        pltpu.VMEM_SHARED((_NP, _H), jnp.float32),
        pltpu.SemaphoreType.DMA,
        pltpu.SemaphoreType.DMA,
    ],
    compiler_params=pltpu.CompilerParams(use_tc_tiling_on_sc=False),
)


def _ep_body(p_ref, b_ref, o_ref):
    full = jnp.concatenate([p_ref[0], p_ref[1]], axis=1)
    bias = jnp.concatenate([b_ref[0], b_ref[1]], axis=1)
    o_ref[...] = full + bias


def _epilogue(parts, bias2d):
    bn = 2000
    return pl.pallas_call(
        _ep_body,
        grid=(_N // bn,),
        in_specs=[
            pl.BlockSpec((_NC, bn, _H), lambda i: (0, i, 0)),
            pl.BlockSpec((_NC, 1, _H), lambda i: (0, 0, 0)),
        ],
        out_specs=pl.BlockSpec((bn, _F), lambda i: (i, 0)),
        out_shape=jax.ShapeDtypeStruct((_N, _F), jnp.float32),
    )(parts, bias2d)


def kernel(x, edge_index, edge_type, weight, h_bias):
    xr = _rel_transform(x, weight).reshape(_NC * _R * _N, _H)
    et2d = edge_type.reshape(_E // _F, _F)
    src2d = edge_index[0].reshape(_E // _F, _F)
    gi = _gather_indices(et2d, src2d).reshape(_NC, _NS, _EPT)
    zeros = jnp.zeros((_NP, _H), jnp.float32)
    dst = edge_index[1].reshape(_NS, _NPH, _EPH // _SCH, _SCH)
    parts = _sc_scatter(xr, gi, dst, zeros)
    return _epilogue(parts, h_bias.reshape(_NC, 1, _H))


# speedup vs baseline: 1.3117x; 1.0445x over previous
import jax
import jax.numpy as jnp
from jax import lax
from jax.experimental import pallas as pl
from jax.experimental.pallas import tpu as pltpu
from jax.experimental.pallas import tpu_sc as plsc

_N = 10000
_E = 320000
_F = 128
_H = _F // 2
_R = 8

_NC = 2
_NS = 16
_EPT = _E // _NS
_NPH = 2
_EPH = _EPT // _NPH
_GCH = 400
_NG = _EPH // _GCH
_SCH = 100
_NSUB = _GCH // _SCH
_NP = 10240
_RPT = _NP // _NS


def _mm_body(x_ref, w_ref, o_ref):
    o_ref[...] = jnp.dot(x_ref[...], w_ref[0],
                         preferred_element_type=jnp.float32)


def _rel_transform(x, weight):
    return pl.pallas_call(
        _mm_body,
        grid=(_R,),
        in_specs=[
            pl.BlockSpec((_N, _F), lambda r: (0, 0)),
            pl.BlockSpec((1, _F, _F), lambda r: (r, 0, 0)),
        ],
        out_specs=pl.BlockSpec((_N, _F), lambda r: (r, 0)),
        out_shape=jax.ShapeDtypeStruct((_R * _N, _F), jnp.float32),
    )(x, weight)


def _gi_body(et_ref, ei_ref, gi_ref, dst_ref):
    blk = _E // _F
    g = (et_ref[0] * _N + ei_ref[0]) * 2
    g2 = g.reshape(blk, _F)
    gi_ref[0] = g2
    gi_ref[1] = g2 + 1
    dst_ref[...] = ei_ref[1].reshape(blk, _F)


def _gather_indices(et_row, edge_index):
    rows = _E // _F
    return pl.pallas_call(
        _gi_body,
        grid=(1,),
        in_specs=[
            pl.BlockSpec((1, _E), lambda i: (0, 0)),
            pl.BlockSpec((2, _E), lambda i: (0, 0)),
        ],
        out_specs=[
            pl.BlockSpec((2, rows, _F), lambda i: (0, 0, 0)),
            pl.BlockSpec((rows, _F), lambda i: (0, 0)),
        ],
        out_shape=[
            jax.ShapeDtypeStruct((2, rows, _F), jnp.int32),
            jax.ShapeDtypeStruct((rows, _F), jnp.int32),
        ],
    )(et_row, edge_index)


def _sc_body(xr_hbm, gi_hbm, dst_hbm, zeros_hbm, out_hbm,
             gi_v, dst_v, rows0_v, rows1_v, agg_s, gsem0, gsem1):
    c = lax.axis_index("c")
    s = lax.axis_index("s")

    rows = pl.ds(s * _RPT, _RPT)
    pltpu.sync_copy(zeros_hbm.at[rows], agg_s.at[rows])

    plsc.subcore_barrier()

    bufs = (rows0_v, rows1_v)
    gsems = (gsem0, gsem1)

    def _gather_start(m, b):
        pltpu.async_copy(xr_hbm.at[gi_v.at[pl.ds(m * _GCH, _GCH)]],
                         bufs[b], gsems[b])

    def _gather_wait(b):
        pltpu.make_async_copy(xr_hbm.at[gi_v.at[pl.ds(0, _GCH)]],
                              bufs[b], gsems[b]).wait()

    for ph in range(_NPH):
        pltpu.sync_copy(gi_hbm.at[c, s, pl.ds(ph * _EPH, _EPH)], gi_v)
        pltpu.sync_copy(dst_hbm.at[s, ph], dst_v)

        _gather_start(0, 0)

        def _pair(t, carry):
            for b in range(2):
                m = t * 2 + b
                _gather_wait(b)
                nxt = m + 1

                @pl.when(nxt < _NG)
                def _():
                    _gather_start(nxt, (b + 1) % 2)

                for k in range(_NSUB):
                    pltpu.sync_copy(
                        bufs[b].at[pl.ds(k * _SCH, _SCH)],
                        agg_s.at[dst_v.at[m * _NSUB + k]], add=True)
            return carry
        lax.fori_loop(0, _NG // 2, _pair, 0)

        if _NG % 2:
            m = _NG - 1
            _gather_wait(0)
            for k in range(_NSUB):
                pltpu.sync_copy(
                    bufs[0].at[pl.ds(k * _SCH, _SCH)],
                    agg_s.at[dst_v.at[m * _NSUB + k]], add=True)

    plsc.subcore_barrier()

    pltpu.sync_copy(agg_s.at[rows], out_hbm.at[c, rows])


_sc_scatter = pl.kernel(
    _sc_body,
    out_type=jax.ShapeDtypeStruct((_NC, _NP, _H), jnp.float32),
    mesh=plsc.VectorSubcoreMesh(core_axis_name="c", subcore_axis_name="s",
                                num_cores=_NC, num_subcores=_NS),
    scratch_types=[
        pltpu.VMEM((_EPH,), jnp.int32),
        pltpu.VMEM((_EPH // _SCH, _SCH), jnp.int32),
        pltpu.VMEM((_GCH, _H), jnp.float32),
        pltpu.VMEM((_GCH, _H), jnp.float32),
        pltpu.VMEM_SHARED((_NP, _H), jnp.float32),
        pltpu.SemaphoreType.DMA,
        pltpu.SemaphoreType.DMA,
    ],
    compiler_params=pltpu.CompilerParams(use_tc_tiling_on_sc=False),
)


def _ep_body(p_ref, b_ref, o_ref):
    full = jnp.concatenate([p_ref[0], p_ref[1]], axis=1)
    bias = jnp.concatenate([b_ref[0], b_ref[1]], axis=1)
    o_ref[...] = full + bias


def _epilogue(parts, bias2d):
    bn = 2000
    return pl.pallas_call(
        _ep_body,
        grid=(_N // bn,),
        in_specs=[
            pl.BlockSpec((_NC, bn, _H), lambda i: (0, i, 0)),
            pl.BlockSpec((_NC, 1, _H), lambda i: (0, 0, 0)),
        ],
        out_specs=pl.BlockSpec((bn, _F), lambda i: (i, 0)),
        out_shape=jax.ShapeDtypeStruct((_N, _F), jnp.float32),
    )(parts, bias2d)


def kernel(x, edge_index, edge_type, weight, h_bias):
    xr = _rel_transform(x, weight).reshape(_NC * _R * _N, _H)
    gi2d, dst2d = _gather_indices(edge_type.reshape(1, _E), edge_index)
    gi = gi2d.reshape(_NC, _NS, _EPT)
    dst = dst2d.reshape(_NS, _NPH, _EPH // _SCH, _SCH)
    zeros = jnp.zeros((_NP, _H), jnp.float32)
    parts = _sc_scatter(xr, gi, dst, zeros)
    return _epilogue(parts, h_bias.reshape(_NC, 1, _H))
